# baseline (speedup 1.0000x reference)
import functools

import jax
import jax.numpy as jnp
from jax import lax
from jax.experimental import pallas as pl
from jax.experimental.pallas import tpu as pltpu
from jax.experimental.pallas import tpu_sc as plsc

D = 16
NC = 2
NS = 16
NW = NC * NS
BATCH = 16384
PER_W = BATCH // NW
WAVE = 16
NWAVE = PER_W // WAVE

_mesh = plsc.VectorSubcoreMesh(core_axis_name="c", subcore_axis_name="s")


@functools.partial(
    pl.kernel,
    mesh=_mesh,
    compiler_params=pltpu.CompilerParams(
        needs_layout_passes=False,
        use_tc_tiling_on_sc=True,
        disable_bounds_checks=True,
    ),
    out_type=jax.ShapeDtypeStruct((BATCH,), jnp.float32),
    scratch_types=[
        pltpu.VMEM((PER_W,), jnp.int32),
        pltpu.VMEM((PER_W,), jnp.int32),
        pltpu.VMEM((WAVE * D, 128), jnp.float32),
        pltpu.VMEM((WAVE * D, 128), jnp.float32),
        pltpu.VMEM((PER_W * D,), jnp.float32),
        pltpu.VMEM((PER_W,), jnp.float32),
        pltpu.VMEM((24, 128), jnp.float32),
        pltpu.SemaphoreType.DMA,
        pltpu.SemaphoreType.DMA,
    ],
)
def _gmf_sc(uidx_hbm, iidx_hbm, ut_hbm, it_hbm, wb_hbm, out_hbm,
            uiv, iiv, buf0, buf1, ucol_v, obuf_v, wb_v, sem0, sem1):
    wid = lax.axis_index("s") * NC + lax.axis_index("c")
    base = wid * PER_W

    pltpu.sync_copy(uidx_hbm.at[pl.ds(base, PER_W)], uiv)
    pltpu.sync_copy(iidx_hbm.at[pl.ds(base, PER_W)], iiv)
    pltpu.sync_copy(wb_hbm, wb_v)

    iot = lax.iota(jnp.int32, D)
    wregs = [wb_v[d, pl.ds(0, D)] for d in range(D)]
    bias = wb_v[D, pl.ds(0, D)]

    def fire(tbl, idx_v, w, buf, sem):
        qv = (idx_v[pl.ds(w * WAVE, WAVE)] >> 7) << 7
        for j in range(WAVE):
            q = pl.multiple_of(qv[j], 128)
            pltpu.async_copy(
                tbl.at[:, pl.ds(q, 128)], buf.at[pl.ds(j * D, D)], sem)

    def drain(tbl, buf, sem):
        for _ in range(WAVE):
            pltpu.make_async_copy(
                tbl.at[:, pl.ds(0, 128)], buf.at[pl.ds(0, D)], sem).wait()

    def consume_u(w, buf):
        uc = uiv[pl.ds(w * WAVE, WAVE)] & 127
        for d in range(D):
            vals = plsc.load_gather(buf, [iot * D + d, uc])
            ucol_v[pl.ds(d * PER_W + w * WAVE, WAVE)] = vals

    def consume_i(w, buf):
        ic = iiv[pl.ds(w * WAVE, WAVE)] & 127
        acc = bias
        for d in range(D):
            iv = plsc.load_gather(buf, [iot * D + d, ic])
            uv = ucol_v[pl.ds(d * PER_W + w * WAVE, WAVE)]
            acc = acc + uv * iv * wregs[d]
        obuf_v[pl.ds(w * WAVE, WAVE)] = jnp.maximum(acc, 0.0)

    def make_pass(tbl, idx_v, consume):
        fire(tbl, idx_v, 0, buf0, sem0)

        def pair(p, last):
            w = 2 * p
            fire(tbl, idx_v, w + 1, buf1, sem1)
            drain(tbl, buf0, sem0)
            consume(w, buf0)
            if not last:
                fire(tbl, idx_v, w + 2, buf0, sem0)
            drain(tbl, buf1, sem1)
            consume(w + 1, buf1)

        def body(p, carry):
            pair(p, False)
            return carry

        lax.fori_loop(0, NWAVE // 2 - 1, body, 0)
        pair(NWAVE // 2 - 1, True)

    make_pass(ut_hbm, uiv, consume_u)
    make_pass(it_hbm, iiv, consume_i)

    pltpu.sync_copy(obuf_v, out_hbm.at[pl.ds(base, PER_W)])


def kernel(user, item, user_table, item_table, W, b):
    u = user.astype(jnp.int32)
    i = item.astype(jnp.int32)
    ut_t = user_table.T
    it_t = item_table.T
    wb = jnp.concatenate(
        [
            jnp.broadcast_to(W.reshape(D, 1), (D, 128)),
            jnp.broadcast_to(b.reshape(1, 1), (1, 128)),
            jnp.zeros((24 - D - 1, 128), jnp.float32),
        ],
        axis=0,
    )
    out = _gmf_sc(u, i, ut_t, it_t, wb)
    return out.reshape(BATCH, 1)
